# baseline (speedup 1.0000x reference)
import jax
import jax.numpy as jnp
from jax import lax
from jax.experimental import pallas as pl
from jax.experimental.pallas import tpu as pltpu

_N = 262144
_C = 64
_NUM_OPS = 16
_NUM_WRITE_OPS = 8

_B = 2048
_G = _N // _B


def _fused_body(wp_ref, op_ref, pred_ref, t_ref, o_ref):
    opv = op_ref[0, 0, :]
    prv = pred_ref[0, 0, :].astype(jnp.float32)
    pos = jnp.full((_B,), _C, dtype=jnp.int32)
    for k in range(_NUM_OPS):
        tgt = jnp.where(k < _NUM_WRITE_OPS, wp_ref[k], _C)
        pos = jnp.where(opv == k, tgt, pos)
    hit = lax.broadcasted_iota(jnp.int32, (_B, _C), 1) == pos[:, None]
    o_ref[...] = jnp.where(hit, prv[:, None], t_ref[...])


def kernel(tensor, operation, prediction, write_positions):
    op3 = operation.reshape(_G, 1, _B)
    pred3 = prediction.reshape(_G, 1, _B)
    return pl.pallas_call(
        _fused_body,
        grid=(_G,),
        in_specs=[
            pl.BlockSpec(memory_space=pltpu.SMEM),
            pl.BlockSpec((1, 1, _B), lambda i: (i, 0, 0)),
            pl.BlockSpec((1, 1, _B), lambda i: (i, 0, 0)),
            pl.BlockSpec((_B, _C), lambda i: (i, 0)),
        ],
        out_specs=pl.BlockSpec((_B, _C), lambda i: (i, 0)),
        out_shape=jax.ShapeDtypeStruct((_N, _C), jnp.float32),
        compiler_params=pltpu.CompilerParams(
            dimension_semantics=("arbitrary",)),
    )(write_positions, op3, pred3, tensor)


# speedup vs baseline: 1.2356x; 1.2356x over previous
import jax
import jax.numpy as jnp
from jax import lax
from jax.experimental import pallas as pl
from jax.experimental.pallas import tpu as pltpu

_N = 262144
_C = 64
_NUM_OPS = 16
_NUM_WRITE_OPS = 8

_B = 16384
_G = _N // _B


def _fused_body(wp_ref, op_ref, pred_ref, t_ref, o_ref):
    opv = op_ref[0, 0, :]
    prv = pred_ref[0, 0, :].astype(jnp.float32)
    pos = jnp.full((_B,), _C, dtype=jnp.int32)
    for k in range(_NUM_OPS):
        tgt = jnp.where(k < _NUM_WRITE_OPS, wp_ref[k], _C)
        pos = jnp.where(opv == k, tgt, pos)
    hit = lax.broadcasted_iota(jnp.int32, (_B, _C), 1) == pos[:, None]
    o_ref[...] = jnp.where(hit, prv[:, None], t_ref[...])


def kernel(tensor, operation, prediction, write_positions):
    op3 = operation.reshape(_G, 1, _B)
    pred3 = prediction.reshape(_G, 1, _B)
    return pl.pallas_call(
        _fused_body,
        grid=(_G,),
        in_specs=[
            pl.BlockSpec(memory_space=pltpu.SMEM),
            pl.BlockSpec((1, 1, _B), lambda i: (i, 0, 0)),
            pl.BlockSpec((1, 1, _B), lambda i: (i, 0, 0)),
            pl.BlockSpec((_B, _C), lambda i: (i, 0)),
        ],
        out_specs=pl.BlockSpec((_B, _C), lambda i: (i, 0)),
        out_shape=jax.ShapeDtypeStruct((_N, _C), jnp.float32),
        compiler_params=pltpu.CompilerParams(
            dimension_semantics=("arbitrary",)),
    )(write_positions, op3, pred3, tensor)
